# baseline (speedup 1.0000x reference)
import functools
import math

import jax
import jax.numpy as jnp
import numpy as np
from jax import lax
from jax.experimental import pallas as pl
from jax.experimental.pallas import tpu as pltpu
from jax.experimental.pallas import tpu_sc as plsc

NTOKENS = 100000
NINP = 128
SEQ = 2048
BATCH = 16
SCALE = math.sqrt(float(NINP))

NROWS = SEQ * BATCH
NW = 32
ROWS_PER_W = NROWS // NW
CHUNK = 128
NCHUNK = ROWS_PER_W // CHUNK
S_PER_W = SEQ // NW
S_PER_CHUNK = CHUNK // BATCH
LANE_GROUPS = NINP // 16


def _make_pe() -> np.ndarray:
    position = np.arange(SEQ, dtype=np.float32)[:, None]
    div_term = np.exp(
        np.arange(0, NINP, 2, dtype=np.float32) * (-math.log(10000.0) / NINP)
    )
    pe = np.zeros((SEQ, NINP), dtype=np.float32)
    pe[:, 0::2] = np.sin(position * div_term)
    pe[:, 1::2] = np.cos(position * div_term)
    return pe


_PE = _make_pe()

_mesh = plsc.VectorSubcoreMesh(core_axis_name="c", subcore_axis_name="s")


@functools.partial(
    pl.kernel,
    mesh=_mesh,
    out_type=jax.ShapeDtypeStruct((NROWS, NINP), jnp.float32),
    scratch_types=[
        pltpu.VMEM((NCHUNK, CHUNK), jnp.int32),
        pltpu.VMEM((S_PER_W, NINP), jnp.float32),
        pltpu.VMEM((CHUNK, NINP), jnp.float32),
        pltpu.SemaphoreType.DMA,
    ],
)
def _sc_embed(x_hbm, pe_hbm, table_hbm, out_hbm, idx_v, pe_v, buf, sem):
    cid = lax.axis_index("c")
    sid = lax.axis_index("s")
    wid = sid * 2 + cid
    base = wid * ROWS_PER_W

    pltpu.sync_copy(x_hbm.at[pl.ds(wid * NCHUNK, NCHUNK)], idx_v)
    pltpu.sync_copy(pe_hbm.at[pl.ds(wid * S_PER_W, S_PER_W)], pe_v)

    for c in range(NCHUNK):
        pltpu.async_copy(table_hbm.at[idx_v.at[c]], buf, sem).wait()

        def s_body(si, _, c=c):
            prow = c * S_PER_CHUNK + si
            pvals = [pe_v[prow, pl.ds(16 * j, 16)] for j in range(LANE_GROUPS)]

            def b_body(b, _):
                r = si * BATCH + b
                for j in range(LANE_GROUPS):
                    buf[r, pl.ds(16 * j, 16)] = (
                        buf[r, pl.ds(16 * j, 16)] * SCALE + pvals[j]
                    )
                return 0

            return lax.fori_loop(0, BATCH, b_body, 0)

        lax.fori_loop(0, S_PER_CHUNK, s_body, 0)

        pltpu.sync_copy(buf, out_hbm.at[pl.ds(base + c * CHUNK, CHUNK)])


def kernel(x, table):
    x2 = x.reshape(NROWS // CHUNK, CHUNK)
    pe = jnp.asarray(_PE)
    out = _sc_embed(x2, pe, table)
    return out.reshape(SEQ, BATCH, NINP)


# speedup vs baseline: 1.7175x; 1.7175x over previous
import functools
import math

import jax
import jax.numpy as jnp
import numpy as np
from jax import lax
from jax.experimental import pallas as pl
from jax.experimental.pallas import tpu as pltpu
from jax.experimental.pallas import tpu_sc as plsc

NTOKENS = 100000
NINP = 128
SEQ = 2048
BATCH = 16
SCALE = math.sqrt(float(NINP))

NROWS = SEQ * BATCH
NW = 32
ROWS_PER_W = NROWS // NW
S_PER_W = SEQ // NW
LANE_GROUPS = NINP // 16

CH = BATCH
NCH = ROWS_PER_W // CH
NBUF = 2
NITER = NCH // NBUF


def _make_pe() -> np.ndarray:
    position = np.arange(SEQ, dtype=np.float32)[:, None]
    div_term = np.exp(
        np.arange(0, NINP, 2, dtype=np.float32) * (-math.log(10000.0) / NINP)
    )
    pe = np.zeros((SEQ, NINP), dtype=np.float32)
    pe[:, 0::2] = np.sin(position * div_term)
    pe[:, 1::2] = np.cos(position * div_term)
    return pe


_PE = _make_pe()

_mesh = plsc.VectorSubcoreMesh(core_axis_name="c", subcore_axis_name="s")


def _compute_chunk(c, gbuf, obuf, pe_v):
    pvals = [pe_v[c, pl.ds(16 * j, 16)] for j in range(LANE_GROUPS)]
    for r in range(CH):
        for j in range(LANE_GROUPS):
            obuf[r, pl.ds(16 * j, 16)] = (
                gbuf[r, pl.ds(16 * j, 16)] * SCALE + pvals[j]
            )


@functools.partial(
    pl.kernel,
    mesh=_mesh,
    out_type=jax.ShapeDtypeStruct((NROWS, NINP), jnp.float32),
    scratch_types=(
        [pltpu.VMEM((NCH, CH), jnp.int32)]
        + [pltpu.VMEM((S_PER_W, NINP), jnp.float32)]
        + [pltpu.VMEM((CH, NINP), jnp.float32)] * NBUF
        + [pltpu.VMEM((CH, NINP), jnp.float32)] * NBUF
        + [pltpu.SemaphoreType.DMA] * (2 * NBUF)
    ),
)
def _sc_embed(x_hbm, pe_hbm, table_hbm, out_hbm, idx_v, pe_v, *rest):
    gbufs = rest[0:NBUF]
    obufs = rest[NBUF : 2 * NBUF]
    gsems = rest[2 * NBUF : 3 * NBUF]
    osems = rest[3 * NBUF : 4 * NBUF]

    wid = lax.axis_index("s") * 2 + lax.axis_index("c")
    base = wid * ROWS_PER_W

    pltpu.sync_copy(x_hbm.at[pl.ds(wid * NCH, NCH)], idx_v)
    pltpu.sync_copy(pe_hbm.at[pl.ds(wid * S_PER_W, S_PER_W)], pe_v)

    prime = [
        pltpu.async_copy(table_hbm.at[idx_v.at[b]], gbufs[b], gsems[b])
        for b in range(NBUF)
    ]

    for b in range(NBUF):
        prime[b].wait()
        _compute_chunk(b, gbufs[b], obufs[b], pe_v)
        pltpu.async_copy(table_hbm.at[idx_v.at[b + NBUF]], gbufs[b], gsems[b])
        pltpu.async_copy(
            obufs[b], out_hbm.at[pl.ds(base + b * CH, CH)], osems[b]
        )

    def g_body(g, carry):
        for b in range(NBUF):
            c = g * NBUF + b
            pltpu.make_async_copy(
                table_hbm.at[idx_v.at[c]], gbufs[b], gsems[b]
            ).wait()
            pltpu.make_async_copy(
                obufs[b], out_hbm.at[pl.ds(base, CH)], osems[b]
            ).wait()
            _compute_chunk(c, gbufs[b], obufs[b], pe_v)

            @pl.when(g < NITER - 1)
            def _issue_next(b=b, c=c):
                pltpu.async_copy(
                    table_hbm.at[idx_v.at[c + NBUF]], gbufs[b], gsems[b]
                )

            pltpu.async_copy(
                obufs[b], out_hbm.at[pl.ds(base + c * CH, CH)], osems[b]
            )
        return carry

    lax.fori_loop(1, NITER, g_body, 0)

    for b in range(NBUF):
        pltpu.make_async_copy(
            obufs[b], out_hbm.at[pl.ds(base, CH)], osems[b]
        ).wait()


def kernel(x, table):
    pe = jnp.asarray(_PE)
    out = _sc_embed(x, pe, table)
    return out.reshape(SEQ, BATCH, NINP)


# speedup vs baseline: 2.2046x; 1.2836x over previous
import functools
import math

import jax
import jax.numpy as jnp
import numpy as np
from jax import lax
from jax.experimental import pallas as pl
from jax.experimental.pallas import tpu as pltpu
from jax.experimental.pallas import tpu_sc as plsc

NTOKENS = 100000
NINP = 128
SEQ = 2048
BATCH = 16
SCALE = math.sqrt(float(NINP))

NROWS = SEQ * BATCH
NW = 32
ROWS_PER_W = NROWS // NW
S_PER_W = SEQ // NW
LANE_GROUPS = NINP // 16

CH = 64
S_PER_CH = CH // BATCH
NCH = ROWS_PER_W // CH
NBUF = 2
NITER = NCH // NBUF


def _make_pe() -> np.ndarray:
    position = np.arange(SEQ, dtype=np.float32)[:, None]
    div_term = np.exp(
        np.arange(0, NINP, 2, dtype=np.float32) * (-math.log(10000.0) / NINP)
    )
    pe = np.zeros((SEQ, NINP), dtype=np.float32)
    pe[:, 0::2] = np.sin(position * div_term)
    pe[:, 1::2] = np.cos(position * div_term)
    return pe


_PE = _make_pe()

_mesh = plsc.VectorSubcoreMesh(core_axis_name="c", subcore_axis_name="s")


def _compute_chunk(c, gbuf, obuf, pe_v):
    for sj in range(S_PER_CH):
        prow = c * S_PER_CH + sj
        pvals = [pe_v[prow, pl.ds(16 * j, 16)] for j in range(LANE_GROUPS)]
        for bb in range(BATCH):
            r = sj * BATCH + bb
            for j in range(LANE_GROUPS):
                obuf[r, pl.ds(16 * j, 16)] = (
                    gbuf[r, pl.ds(16 * j, 16)] * SCALE + pvals[j]
                )


@functools.partial(
    pl.kernel,
    mesh=_mesh,
    out_type=jax.ShapeDtypeStruct((NROWS, NINP), jnp.float32),
    scratch_types=(
        [pltpu.VMEM((NCH, CH), jnp.int32)]
        + [pltpu.VMEM((S_PER_W, NINP), jnp.float32)]
        + [pltpu.VMEM((CH, NINP), jnp.float32)] * NBUF
        + [pltpu.VMEM((CH, NINP), jnp.float32)] * NBUF
        + [pltpu.SemaphoreType.DMA] * (2 * NBUF)
    ),
)
def _sc_embed(x_hbm, pe_hbm, table_hbm, out_hbm, idx_v, pe_v, *rest):
    gbufs = rest[0:NBUF]
    obufs = rest[NBUF : 2 * NBUF]
    gsems = rest[2 * NBUF : 3 * NBUF]
    osems = rest[3 * NBUF : 4 * NBUF]

    wid = lax.axis_index("s") * 2 + lax.axis_index("c")
    base = wid * ROWS_PER_W

    pltpu.sync_copy(x_hbm.at[pl.ds(wid * NCH, NCH)], idx_v)
    pltpu.sync_copy(pe_hbm.at[pl.ds(wid * S_PER_W, S_PER_W)], pe_v)

    prime = [
        pltpu.async_copy(table_hbm.at[idx_v.at[b]], gbufs[b], gsems[b])
        for b in range(NBUF)
    ]

    for b in range(NBUF):
        prime[b].wait()
        _compute_chunk(b, gbufs[b], obufs[b], pe_v)
        pltpu.async_copy(table_hbm.at[idx_v.at[b + NBUF]], gbufs[b], gsems[b])
        pltpu.async_copy(
            obufs[b], out_hbm.at[pl.ds(base + b * CH, CH)], osems[b]
        )

    def g_body(g, carry):
        for b in range(NBUF):
            c = g * NBUF + b
            pltpu.make_async_copy(
                table_hbm.at[idx_v.at[c]], gbufs[b], gsems[b]
            ).wait()
            pltpu.make_async_copy(
                obufs[b], out_hbm.at[pl.ds(base, CH)], osems[b]
            ).wait()
            _compute_chunk(c, gbufs[b], obufs[b], pe_v)

            @pl.when(g < NITER - 1)
            def _issue_next(b=b, c=c):
                pltpu.async_copy(
                    table_hbm.at[idx_v.at[c + NBUF]], gbufs[b], gsems[b]
                )

            pltpu.async_copy(
                obufs[b], out_hbm.at[pl.ds(base + c * CH, CH)], osems[b]
            )
        return carry

    lax.fori_loop(1, NITER, g_body, 0)

    for b in range(NBUF):
        pltpu.make_async_copy(
            obufs[b], out_hbm.at[pl.ds(base, CH)], osems[b]
        ).wait()


def kernel(x, table):
    x2 = x.reshape(NROWS // CH, CH)
    pe = jnp.asarray(_PE)
    out = _sc_embed(x2, pe, table)
    return out.reshape(SEQ, BATCH, NINP)
